# baseline (speedup 1.0000x reference)
import jax
import jax.numpy as jnp
from jax import lax
from jax.experimental import pallas as pl
from jax.experimental.pallas import tpu as pltpu
from jax.experimental.pallas import tpu_sc as plsc

N = 10000
E = 320000
B = 8
NC = 2
NS = 16
L = 16

NPAD = 10240
ROWS_PT = NPAD // NS
BH = B // NC
C = 4000
M = E // NS
NCHUNK = M // C
EU = 10


def _body(in_h, srcp, dstp, wp, adjp, i0p, swp, bp, out, **scr):
    accs = [scr[f"acc{j}"] for j in range(BH)]
    ins = [scr[f"in{j}"] for j in range(BH)]
    gss = [[scr[f"g{k}_{j}"] for j in range(BH)] for k in range(2)]
    avs = [scr[f"a{j}"] for j in range(BH)]
    srcs = [scr[f"src{k}"] for k in range(2)]
    ws = [scr[f"w{k}"] for k in range(2)]
    adjs = [scr[f"adj{k}"] for k in range(2)]
    dsts = [scr[f"dst{k}"] for k in range(3)]
    lsem = [scr[f"lsem{k}"] for k in range(2)]
    ssem = [scr[f"ssem{k}"] for k in range(3)]
    i0_v, sw_v, b_v = scr["i0_v"], scr["sw_v"], scr["b_v"]

    c = lax.axis_index("c")
    s = lax.axis_index("s")
    zeros = jnp.zeros((L,), jnp.float32)

    def load_descs(ch):
        off = s * M + ch * C
        k = ch % 2
        return [
            pltpu.make_async_copy(srcp.at[pl.ds(off, C)], srcs[k], lsem[k]),
            pltpu.make_async_copy(dstp.at[pl.ds(off, C)], dsts[ch % 3], lsem[k]),
            pltpu.make_async_copy(wp.at[pl.ds(off, C)], ws[k], lsem[k]),
            pltpu.make_async_copy(adjp.at[pl.ds(off, C)], adjs[k], lsem[k]),
        ]

    def scatter_descs(ch):
        k = ch % 2
        return [
            pltpu.make_async_copy(gss[k][j], accs[j].at[dsts[ch % 3]],
                                  ssem[ch % 3])
            for j in range(BH)
        ]

    for d in load_descs(0):
        d.start()
    for j in range(BH):
        pltpu.sync_copy(in_h.at[pl.ds((c * BH + j) * N, N)],
                        ins[j].at[pl.ds(0, N)])

    def zero_body(v, _):
        i0_v[pl.ds(v * L, L)] = zeros
        return _
    lax.fori_loop(0, ROWS_PT // L, zero_body, None)
    for j in range(BH):
        pltpu.sync_copy(i0_v, accs[j].at[pl.ds(s * ROWS_PT, ROWS_PT)])
    plsc.subcore_barrier()

    for ch in range(NCHUNK):
        k = ch % 2
        for d in load_descs(ch):
            d.wait()
        if ch + 1 < NCHUNK:
            for d in load_descs(ch + 1):
                d.start()
        if ch >= 2:
            for d in scatter_descs(ch - 2):
                d.wait()

        def edge_body(g, _):
            for u in range(EU):
                sl = pl.ds(g * (EU * L) + u * L, L)
                s16 = srcs[k][sl]
                w16 = ws[k][sl] * adjs[k][sl]
                for j in range(BH):
                    gss[k][j][sl] = plsc.load_gather(ins[j], [s16]) * w16
            return _
        lax.fori_loop(0, C // (EU * L), edge_body, None)

        for d in scatter_descs(ch):
            d.start(add=True)

    for d in scatter_descs(NCHUNK - 2):
        d.wait()
    for d in scatter_descs(NCHUNK - 1):
        d.wait()
    plsc.subcore_barrier()

    n0 = s * ROWS_PT
    TAIL = N - (NS - 1) * ROWS_PT
    for j in range(BH):
        pltpu.sync_copy(accs[j].at[pl.ds(n0, ROWS_PT)], avs[j])

    last = s == NS - 1

    @pl.when(jnp.logical_not(last))
    def _():
        pltpu.sync_copy(i0p.at[pl.ds(n0, ROWS_PT)], i0_v)
        pltpu.sync_copy(swp.at[pl.ds(n0, ROWS_PT)], sw_v)
        pltpu.sync_copy(bp.at[pl.ds(n0, ROWS_PT)], b_v)

    @pl.when(last)
    def _():
        pltpu.sync_copy(i0p.at[pl.ds(n0, TAIL)], i0_v.at[pl.ds(0, TAIL)])
        pltpu.sync_copy(swp.at[pl.ds(n0, TAIL)], sw_v.at[pl.ds(0, TAIL)])
        pltpu.sync_copy(bp.at[pl.ds(n0, TAIL)], b_v.at[pl.ds(0, TAIL)])

    def fin(v, _):
        sl = pl.ds(v * L, L)
        sw16 = sw_v[sl] * i0_v[sl]
        b16 = b_v[sl]
        for j in range(BH):
            avs[j][sl] = jnp.maximum(avs[j][sl] * sw16 + b16, 0.0)
        return _
    lax.fori_loop(0, ROWS_PT // L, fin, None)

    @pl.when(jnp.logical_not(last))
    def _():
        for j in range(BH):
            pltpu.sync_copy(avs[j], out.at[pl.ds((c * BH + j) * N + n0, ROWS_PT)])

    @pl.when(last)
    def _():
        for j in range(BH):
            pltpu.sync_copy(avs[j].at[pl.ds(0, TAIL)],
                            out.at[pl.ds((c * BH + j) * N + n0, TAIL)])


@jax.jit
def kernel(inputs, adj_values, w, self_w, b, edge_index):
    srcp = edge_index[:, 0]
    dstp = edge_index[:, 1]
    wp = w
    adjp = adj_values

    in_h = inputs.reshape(B * N)
    i0p = inputs[0]
    swp = self_w
    bp = b

    scratch = {}
    for j in range(BH):
        scratch[f"acc{j}"] = pltpu.VMEM_SHARED((NPAD,), jnp.float32)
        scratch[f"in{j}"] = pltpu.VMEM((NPAD,), jnp.float32)
        scratch[f"a{j}"] = pltpu.VMEM((ROWS_PT,), jnp.float32)
        for k in range(2):
            scratch[f"g{k}_{j}"] = pltpu.VMEM((C,), jnp.float32)
    for k in range(2):
        scratch[f"src{k}"] = pltpu.VMEM((C,), jnp.int32)
        scratch[f"w{k}"] = pltpu.VMEM((C,), jnp.float32)
        scratch[f"adj{k}"] = pltpu.VMEM((C,), jnp.float32)
        scratch[f"lsem{k}"] = pltpu.SemaphoreType.DMA
    for k in range(3):
        scratch[f"dst{k}"] = pltpu.VMEM((C,), jnp.int32)
        scratch[f"ssem{k}"] = pltpu.SemaphoreType.DMA
    scratch["i0_v"] = pltpu.VMEM((ROWS_PT,), jnp.float32)
    scratch["sw_v"] = pltpu.VMEM((ROWS_PT,), jnp.float32)
    scratch["b_v"] = pltpu.VMEM((ROWS_PT,), jnp.float32)

    mesh = plsc.VectorSubcoreMesh(
        core_axis_name="c", subcore_axis_name="s",
        num_cores=NC, num_subcores=NS)
    res = pl.kernel(
        _body,
        out_type=jax.ShapeDtypeStruct((B * N,), jnp.float32),
        mesh=mesh,
        scratch_types=scratch,
        compiler_params=pltpu.CompilerParams(needs_layout_passes=False),
        name="gnn_layer_sc",
    )(in_h, srcp, dstp, wp, adjp, i0p, swp, bp)

    return res.reshape(B, N)


# speedup vs baseline: 1.0528x; 1.0528x over previous
import jax
import jax.numpy as jnp
from jax import lax
from jax.experimental import pallas as pl
from jax.experimental.pallas import tpu as pltpu
from jax.experimental.pallas import tpu_sc as plsc

N = 10000
E = 320000
B = 8
NC = 2
NS = 16
L = 16

NPAD = 10240
ROWS_PT = NPAD // NS
BH = B // NC
C = 4000
M = E // NS
NCHUNK = M // C
EU = 5


def _body(in_h, srcp, dstp, wp, adjp, i0p, swp, bp, out, **scr):
    accs = [scr[f"acc{j}"] for j in range(BH)]
    ins = [scr[f"in{j}"] for j in range(BH)]
    gss = [[scr[f"g{k}_{j}"] for j in range(BH)] for k in range(2)]
    avs = [scr[f"a{j}"] for j in range(BH)]
    srcs = [scr[f"src{k}"] for k in range(2)]
    ws = [scr[f"w{k}"] for k in range(2)]
    adjs = [scr[f"adj{k}"] for k in range(2)]
    dsts = [scr[f"dst{k}"] for k in range(3)]
    lsem = [scr[f"lsem{k}"] for k in range(2)]
    ssem = [scr[f"ssem{k}"] for k in range(3)]
    i0_v, sw_v, b_v = scr["i0_v"], scr["sw_v"], scr["b_v"]

    c = lax.axis_index("c")
    s = lax.axis_index("s")
    zeros = jnp.zeros((L,), jnp.float32)

    def load_descs(ch):
        off = s * M + ch * C
        k = ch % 2
        return [
            pltpu.make_async_copy(srcp.at[pl.ds(off, C)], srcs[k], lsem[k]),
            pltpu.make_async_copy(dstp.at[pl.ds(off, C)], dsts[ch % 3], lsem[k]),
            pltpu.make_async_copy(wp.at[pl.ds(off, C)], ws[k], lsem[k]),
            pltpu.make_async_copy(adjp.at[pl.ds(off, C)], adjs[k], lsem[k]),
        ]

    def scatter_descs(ch):
        k = ch % 2
        return [
            pltpu.make_async_copy(gss[k][j], accs[j].at[dsts[ch % 3]],
                                  ssem[ch % 3])
            for j in range(BH)
        ]

    for d in load_descs(0):
        d.start()
    for j in range(BH):
        pltpu.sync_copy(in_h.at[pl.ds((c * BH + j) * N, N)],
                        ins[j].at[pl.ds(0, N)])

    def zero_body(v, _):
        i0_v[pl.ds(v * L, L)] = zeros
        return _
    lax.fori_loop(0, ROWS_PT // L, zero_body, None)
    for j in range(BH):
        pltpu.sync_copy(i0_v, accs[j].at[pl.ds(s * ROWS_PT, ROWS_PT)])
    plsc.subcore_barrier()

    for ch in range(NCHUNK):
        k = ch % 2
        for d in load_descs(ch):
            d.wait()
        if ch + 1 < NCHUNK:
            for d in load_descs(ch + 1):
                d.start()
        if ch >= 2:
            for d in scatter_descs(ch - 2):
                d.wait()

        def edge_body(g, _):
            for u in range(EU):
                sl = pl.ds(g * (EU * L) + u * L, L)
                s16 = srcs[k][sl]
                w16 = ws[k][sl] * adjs[k][sl]
                for j in range(BH):
                    gss[k][j][sl] = plsc.load_gather(ins[j], [s16]) * w16
            return _
        lax.fori_loop(0, C // (EU * L), edge_body, None)

        for d in scatter_descs(ch):
            d.start(add=True)

    for d in scatter_descs(NCHUNK - 2):
        d.wait()
    for d in scatter_descs(NCHUNK - 1):
        d.wait()
    plsc.subcore_barrier()

    n0 = s * ROWS_PT
    TAIL = N - (NS - 1) * ROWS_PT
    for j in range(BH):
        pltpu.sync_copy(accs[j].at[pl.ds(n0, ROWS_PT)], avs[j])

    last = s == NS - 1

    @pl.when(jnp.logical_not(last))
    def _():
        pltpu.sync_copy(i0p.at[pl.ds(n0, ROWS_PT)], i0_v)
        pltpu.sync_copy(swp.at[pl.ds(n0, ROWS_PT)], sw_v)
        pltpu.sync_copy(bp.at[pl.ds(n0, ROWS_PT)], b_v)

    @pl.when(last)
    def _():
        pltpu.sync_copy(i0p.at[pl.ds(n0, TAIL)], i0_v.at[pl.ds(0, TAIL)])
        pltpu.sync_copy(swp.at[pl.ds(n0, TAIL)], sw_v.at[pl.ds(0, TAIL)])
        pltpu.sync_copy(bp.at[pl.ds(n0, TAIL)], b_v.at[pl.ds(0, TAIL)])

    def fin(v, _):
        sl = pl.ds(v * L, L)
        sw16 = sw_v[sl] * i0_v[sl]
        b16 = b_v[sl]
        for j in range(BH):
            avs[j][sl] = jnp.maximum(avs[j][sl] * sw16 + b16, 0.0)
        return _
    lax.fori_loop(0, ROWS_PT // L, fin, None)

    @pl.when(jnp.logical_not(last))
    def _():
        for j in range(BH):
            pltpu.sync_copy(avs[j], out.at[pl.ds((c * BH + j) * N + n0, ROWS_PT)])

    @pl.when(last)
    def _():
        for j in range(BH):
            pltpu.sync_copy(avs[j].at[pl.ds(0, TAIL)],
                            out.at[pl.ds((c * BH + j) * N + n0, TAIL)])


@jax.jit
def kernel(inputs, adj_values, w, self_w, b, edge_index):
    srcp = edge_index[:, 0]
    dstp = edge_index[:, 1]
    wp = w
    adjp = adj_values

    in_h = inputs.reshape(B * N)
    i0p = inputs[0]
    swp = self_w
    bp = b

    scratch = {}
    for j in range(BH):
        scratch[f"acc{j}"] = pltpu.VMEM_SHARED((NPAD,), jnp.float32)
        scratch[f"in{j}"] = pltpu.VMEM((NPAD,), jnp.float32)
        scratch[f"a{j}"] = pltpu.VMEM((ROWS_PT,), jnp.float32)
        for k in range(2):
            scratch[f"g{k}_{j}"] = pltpu.VMEM((C,), jnp.float32)
    for k in range(2):
        scratch[f"src{k}"] = pltpu.VMEM((C,), jnp.int32)
        scratch[f"w{k}"] = pltpu.VMEM((C,), jnp.float32)
        scratch[f"adj{k}"] = pltpu.VMEM((C,), jnp.float32)
        scratch[f"lsem{k}"] = pltpu.SemaphoreType.DMA
    for k in range(3):
        scratch[f"dst{k}"] = pltpu.VMEM((C,), jnp.int32)
        scratch[f"ssem{k}"] = pltpu.SemaphoreType.DMA
    scratch["i0_v"] = pltpu.VMEM((ROWS_PT,), jnp.float32)
    scratch["sw_v"] = pltpu.VMEM((ROWS_PT,), jnp.float32)
    scratch["b_v"] = pltpu.VMEM((ROWS_PT,), jnp.float32)

    mesh = plsc.VectorSubcoreMesh(
        core_axis_name="c", subcore_axis_name="s",
        num_cores=NC, num_subcores=NS)
    res = pl.kernel(
        _body,
        out_type=jax.ShapeDtypeStruct((B * N,), jnp.float32),
        mesh=mesh,
        scratch_types=scratch,
        compiler_params=pltpu.CompilerParams(needs_layout_passes=False),
        name="gnn_layer_sc",
    )(in_h, srcp, dstp, wp, adjp, i0p, swp, bp)

    return res.reshape(B, N)
